# baseline (speedup 1.0000x reference)
import jax
import jax.numpy as jnp
from jax.experimental import pallas as pl
from jax.experimental.pallas import tpu as pltpu

TM = 512


def _router_block(x_ref, wt_ref, b_ref, out_ref):
    logits = jnp.dot(x_ref[...], wt_ref[...], preferred_element_type=jnp.float32)
    logits = logits + b_ref[...]
    m = jnp.max(logits, axis=-1, keepdims=True)
    e = jnp.exp(logits - m)
    out_ref[...] = e / jnp.sum(e, axis=-1, keepdims=True)


def kernel(x, W, b):
    tokens, d_model = x.shape
    num_experts = W.shape[0]
    wt = W.T
    b2 = b.reshape(1, num_experts)
    grid = (tokens // TM,)
    return pl.pallas_call(
        _router_block,
        grid=grid,
        in_specs=[
            pl.BlockSpec((TM, d_model), lambda i: (i, 0)),
            pl.BlockSpec((d_model, num_experts), lambda i: (0, 0)),
            pl.BlockSpec((1, num_experts), lambda i: (0, 0)),
        ],
        out_specs=pl.BlockSpec((TM, num_experts), lambda i: (i, 0)),
        out_shape=jax.ShapeDtypeStruct((tokens, num_experts), jnp.float32),
        compiler_params=pltpu.CompilerParams(
            dimension_semantics=("arbitrary",),
        ),
    )(x, wt, b2)


# speedup vs baseline: 1.0194x; 1.0194x over previous
import jax
import jax.numpy as jnp
from jax.experimental import pallas as pl
from jax.experimental.pallas import tpu as pltpu

TM = 1024


def _router_block(x_ref, wt_ref, b_ref, out_ref):
    logits = jnp.dot(x_ref[...], wt_ref[...], preferred_element_type=jnp.float32)
    logits = logits + b_ref[...]
    m = jnp.max(logits, axis=-1, keepdims=True)
    e = jnp.exp(logits - m)
    out_ref[...] = e / jnp.sum(e, axis=-1, keepdims=True)


def kernel(x, W, b):
    tokens, d_model = x.shape
    num_experts = W.shape[0]
    wt = W.T
    b2 = b.reshape(1, num_experts)
    grid = (tokens // TM,)
    return pl.pallas_call(
        _router_block,
        grid=grid,
        in_specs=[
            pl.BlockSpec((TM, d_model), lambda i: (i, 0)),
            pl.BlockSpec((d_model, num_experts), lambda i: (0, 0)),
            pl.BlockSpec((1, num_experts), lambda i: (0, 0)),
        ],
        out_specs=pl.BlockSpec((TM, num_experts), lambda i: (i, 0)),
        out_shape=jax.ShapeDtypeStruct((tokens, num_experts), jnp.float32),
        compiler_params=pltpu.CompilerParams(
            dimension_semantics=("arbitrary",),
        ),
    )(x, wt, b2)
